# Initial kernel scaffold; baseline (speedup 1.0000x reference)
import jax
import jax.numpy as jnp
from jax import lax
from jax.experimental import pallas as pl
from jax.experimental.pallas import tpu as pltpu
from jax.experimental.pallas import tpu_sc as plsc

N_NODES = 10000
D = 128
E = 320000
N_LAYERS = 3

NC = 2
NS = 16
HALF = D // NC
RPT = 640
N_PAD = NS * RPT
C = 128
EPS = E // NS
NCHUNK = -(-EPS // C)
EPS_PAD = NCHUNK * C
LANES = 16


def _sc_body(ego0_h, col_h, row_h, val_h,
             out_h, egoa_h, egob_h,
             colv, rowv, valv, gbuf, tmp, sumv, acc, sem):
    c = lax.axis_index("c")
    s = lax.axis_index("s")
    base = s * RPT
    zero16f = jnp.zeros((LANES,), jnp.float32)

    def _zero_tmp(r, carry):
        for d in range(HALF // LANES):
            tmp[r, pl.ds(d * LANES, LANES)] = zero16f
        return carry

    lax.fori_loop(0, RPT, _zero_tmp, 0)
    pltpu.sync_copy(tmp, acc.at[pl.ds(base, RPT)])

    def _zero_sum(r, carry):
        for d in range(HALF // LANES):
            sumv[r, pl.ds(d * LANES, LANES)] = zero16f
        return carry

    lax.fori_loop(0, RPT, _zero_sum, 0)
    plsc.subcore_barrier()

    col_off = c * N_PAD

    for k in range(N_LAYERS):
        src = (ego0_h, egoa_h, egob_h)[k]

        def _chunk(j, carry):
            pltpu.sync_copy(col_h.at[s, j], colv)
            pltpu.sync_copy(row_h.at[s, j], rowv)
            pltpu.sync_copy(val_h.at[s, j], valv)
            for i in range(C // LANES):
                sl = pl.ds(i * LANES, LANES)
                colv[sl] = colv[sl] + col_off
            pltpu.async_copy(src.at[colv], gbuf, sem).wait()

            def _scale(e, cc):
                v = valv[e]
                for d in range(HALF // LANES):
                    sl = pl.ds(d * LANES, LANES)
                    gbuf[e, sl] = gbuf[e, sl] * v
                return cc

            lax.fori_loop(0, C, _scale, 0)
            pltpu.sync_copy(gbuf, acc.at[rowv], add=True)
            return carry

        lax.fori_loop(0, NCHUNK, _chunk, 0)
        plsc.subcore_barrier()

        pltpu.sync_copy(acc.at[pl.ds(base, RPT)], tmp)
        if k + 1 < N_LAYERS:
            dst = (egoa_h, egob_h)[k]
            pltpu.sync_copy(tmp, dst.at[pl.ds(c * N_PAD + base, RPT)])

        def _accum(r, carry):
            for d in range(HALF // LANES):
                sl = pl.ds(d * LANES, LANES)
                sumv[r, sl] = sumv[r, sl] + tmp[r, sl]
            return carry

        lax.fori_loop(0, RPT, _accum, 0)
        lax.fori_loop(0, RPT, _zero_tmp, 0)
        pltpu.sync_copy(tmp, acc.at[pl.ds(base, RPT)])
        plsc.subcore_barrier()

    inv = jnp.float32(1.0 / N_LAYERS)

    def _mean(r, carry):
        for d in range(HALF // LANES):
            sl = pl.ds(d * LANES, LANES)
            sumv[r, sl] = sumv[r, sl] * inv
        return carry

    lax.fori_loop(0, RPT, _mean, 0)
    pltpu.sync_copy(sumv, out_h.at[c, pl.ds(base, RPT)])


@jax.jit
def _run(ego0, col, row, val):
    mesh = plsc.VectorSubcoreMesh(core_axis_name="c", subcore_axis_name="s",
                                  num_cores=NC, num_subcores=NS)
    f = pl.kernel(
        _sc_body,
        out_type=(
            jax.ShapeDtypeStruct((NC, N_PAD, HALF), jnp.float32),
            jax.ShapeDtypeStruct((NC * N_PAD, HALF), jnp.float32),
            jax.ShapeDtypeStruct((NC * N_PAD, HALF), jnp.float32),
        ),
        mesh=mesh,
        scratch_types=[
            pltpu.VMEM((C,), jnp.int32),
            pltpu.VMEM((C,), jnp.int32),
            pltpu.VMEM((C,), jnp.float32),
            pltpu.VMEM((C, HALF), jnp.float32),
            pltpu.VMEM((RPT, HALF), jnp.float32),
            pltpu.VMEM((RPT, HALF), jnp.float32),
            pltpu.VMEM_SHARED((N_PAD, HALF), jnp.float32),
            pltpu.SemaphoreType.DMA,
        ],
    )
    out, _, _ = f(ego0, col, row, val)
    return out


def kernel(user_emb, item_emb, edge_vals, edge_index):
    ego0 = jnp.concatenate([user_emb, item_emb], axis=0)
    ego0 = jnp.pad(ego0, ((0, N_PAD - N_NODES), (0, 0)))
    ego0 = ego0.reshape(N_PAD, NC, HALF).transpose(1, 0, 2).reshape(NC * N_PAD, HALF)

    pad = NS * EPS_PAD - E
    col = jnp.pad(edge_index[1], (0, pad)).reshape(NS, NCHUNK, C)
    row = jnp.pad(edge_index[0], (0, pad)).reshape(NS, NCHUNK, C)
    val = jnp.pad(edge_vals, (0, pad)).reshape(NS, NCHUNK, C)

    out = _run(ego0, col, row, val)
    final = out[:, :N_NODES, :].transpose(1, 0, 2).reshape(N_NODES, D)
    return (final[:N_NODES // 2], final[N_NODES // 2:])


# speedup vs baseline: 1.9312x; 1.9312x over previous
import jax
import jax.numpy as jnp
from jax import lax
from jax.experimental import pallas as pl
from jax.experimental.pallas import tpu as pltpu
from jax.experimental.pallas import tpu_sc as plsc

N_NODES = 10000
D = 128
E = 320000
N_LAYERS = 3

NC = 2
NS = 16
HALF = D // NC
RPT = 640
N_PAD = NS * RPT
C = 128
EPS = E // NS
NCHUNK = -(-EPS // C)
EPS_PAD = NCHUNK * C
LANES = 16
RC = 128


def _sc_body(ego0_h, col_h, row_h, val_h,
             out_h, egoa_h, egob_h,
             colv, rowv, valv, gbuf, tmp, sumv, acc, sem):
    c = lax.axis_index("c")
    s = lax.axis_index("s")
    base = s * RPT
    zero16f = jnp.zeros((LANES,), jnp.float32)

    def _zero_tmp(r, carry):
        for d in range(HALF // LANES):
            tmp[r, pl.ds(d * LANES, LANES)] = zero16f
        return carry

    lax.fori_loop(0, RC, _zero_tmp, 0)

    def _zero_acc(q, carry):
        pltpu.sync_copy(tmp, acc.at[pl.ds(base + q * RC, RC)])
        return carry

    lax.fori_loop(0, RPT // RC, _zero_acc, 0)

    def _zero_sum(r, carry):
        for d in range(HALF // LANES):
            sumv[r, pl.ds(d * LANES, LANES)] = zero16f
        return carry

    lax.fori_loop(0, RPT, _zero_sum, 0)
    plsc.subcore_barrier()

    col_off = c * N_PAD

    for k in range(N_LAYERS):
        src = (ego0_h, egoa_h, egob_h)[k]

        def _chunk(j, carry):
            pltpu.sync_copy(col_h.at[s, j], colv)
            pltpu.sync_copy(row_h.at[s, j], rowv)
            pltpu.sync_copy(val_h.at[s, j], valv)
            for i in range(C // LANES):
                sl = pl.ds(i * LANES, LANES)
                colv[sl] = colv[sl] + col_off
            pltpu.async_copy(src.at[colv], gbuf, sem).wait()

            def _scale(g, cc):
                e0 = g * LANES
                v16 = valv[pl.ds(e0, LANES)]
                for j in range(LANES):
                    v = v16[j]
                    for d in range(HALF // LANES):
                        sl = pl.ds(d * LANES, LANES)
                        gbuf[e0 + j, sl] = gbuf[e0 + j, sl] * v
                return cc

            lax.fori_loop(0, C // LANES, _scale, 0)
            pltpu.sync_copy(gbuf, acc.at[rowv], add=True)
            return carry

        lax.fori_loop(0, NCHUNK, _chunk, 0)
        plsc.subcore_barrier()

        def _readout(q, carry):
            rb = base + q * RC
            pltpu.sync_copy(acc.at[pl.ds(rb, RC)], tmp)
            if k + 1 < N_LAYERS:
                dst = (egoa_h, egob_h)[k]
                pltpu.sync_copy(tmp, dst.at[pl.ds(c * N_PAD + rb, RC)])

            def _accum(r, carry2):
                for d in range(HALF // LANES):
                    sl = pl.ds(d * LANES, LANES)
                    sumv[q * RC + r, sl] = sumv[q * RC + r, sl] + tmp[r, sl]
                return carry2

            lax.fori_loop(0, RC, _accum, 0)
            lax.fori_loop(0, RC, _zero_tmp, 0)
            pltpu.sync_copy(tmp, acc.at[pl.ds(rb, RC)])
            return carry

        lax.fori_loop(0, RPT // RC, _readout, 0)
        plsc.subcore_barrier()

    inv = jnp.float32(1.0 / N_LAYERS)

    def _mean(r, carry):
        for d in range(HALF // LANES):
            sl = pl.ds(d * LANES, LANES)
            sumv[r, sl] = sumv[r, sl] * inv
        return carry

    lax.fori_loop(0, RPT, _mean, 0)
    pltpu.sync_copy(sumv, out_h.at[c, pl.ds(base, RPT)])


@jax.jit
def _run(ego0, col, row, val):
    mesh = plsc.VectorSubcoreMesh(core_axis_name="c", subcore_axis_name="s",
                                  num_cores=NC, num_subcores=NS)
    f = pl.kernel(
        _sc_body,
        out_type=(
            jax.ShapeDtypeStruct((NC, N_PAD, HALF), jnp.float32),
            jax.ShapeDtypeStruct((NC * N_PAD, HALF), jnp.float32),
            jax.ShapeDtypeStruct((NC * N_PAD, HALF), jnp.float32),
        ),
        mesh=mesh,
        compiler_params=pltpu.CompilerParams(use_tc_tiling_on_sc=False),
        scratch_types=[
            pltpu.VMEM((C,), jnp.int32),
            pltpu.VMEM((C,), jnp.int32),
            pltpu.VMEM((C,), jnp.float32),
            pltpu.VMEM((C, HALF), jnp.float32),
            pltpu.VMEM((RC, HALF), jnp.float32),
            pltpu.VMEM((RPT, HALF), jnp.float32),
            pltpu.VMEM_SHARED((N_PAD, HALF), jnp.float32),
            pltpu.SemaphoreType.DMA,
        ],
    )
    out, _, _ = f(ego0, col, row, val)
    return out


def kernel(user_emb, item_emb, edge_vals, edge_index):
    ego0 = jnp.concatenate([user_emb, item_emb], axis=0)
    ego0 = jnp.pad(ego0, ((0, N_PAD - N_NODES), (0, 0)))
    ego0 = ego0.reshape(N_PAD, NC, HALF).transpose(1, 0, 2).reshape(NC * N_PAD, HALF)

    pad = NS * EPS_PAD - E
    col = jnp.pad(edge_index[1], (0, pad)).reshape(NS, NCHUNK, C)
    row = jnp.pad(edge_index[0], (0, pad)).reshape(NS, NCHUNK, C)
    val = jnp.pad(edge_vals, (0, pad)).reshape(NS, NCHUNK, C)

    out = _run(ego0, col, row, val)
    final = out[:, :N_NODES, :].transpose(1, 0, 2).reshape(N_NODES, D)
    return (final[:N_NODES // 2], final[N_NODES // 2:])


# speedup vs baseline: 2.8849x; 1.4938x over previous
import jax
import jax.numpy as jnp
from jax import lax
from jax.experimental import pallas as pl
from jax.experimental.pallas import tpu as pltpu
from jax.experimental.pallas import tpu_sc as plsc

N_NODES = 10000
D = 128
E = 320000
N_LAYERS = 3

NC = 2
NS = 16
HALF = D // NC
RPT = 640
N_PAD = NS * RPT
C = 128
G = 8
NBUF = 4
EPS = E // NS
NSUPER = -(-EPS // (G * C))
EPS_PAD = NSUPER * G * C
LANES = 16
RC = 128


def _sc_body(ego0_h, col_h, row_h, val_h,
             out_h, egoa_h, egob_h,
             col2, row2, val2, gbufs, tmp, sumv, acc,
             stage_sem, gsems, ssems):
    c = lax.axis_index("c")
    s = lax.axis_index("s")
    base = s * RPT
    zero16f = jnp.zeros((LANES,), jnp.float32)

    def _zero_tmp(r, carry):
        for d in range(HALF // LANES):
            tmp[r, pl.ds(d * LANES, LANES)] = zero16f
        return carry

    lax.fori_loop(0, RC, _zero_tmp, 0)

    def _zero_acc(q, carry):
        pltpu.sync_copy(tmp, acc.at[pl.ds(base + q * RC, RC)])
        return carry

    lax.fori_loop(0, RPT // RC, _zero_acc, 0)

    def _zero_sum(r, carry):
        for d in range(HALF // LANES):
            sumv[r, pl.ds(d * LANES, LANES)] = zero16f
        return carry

    lax.fori_loop(0, RPT, _zero_sum, 0)
    plsc.subcore_barrier()

    col_off = c * N_PAD

    for k in range(N_LAYERS):
        src = (ego0_h, egoa_h, egob_h)[k]

        def _super(jsc, carry):
            a1 = pltpu.async_copy(col_h.at[s, jsc], col2, stage_sem)
            a2 = pltpu.async_copy(row_h.at[s, jsc], row2, stage_sem)
            a3 = pltpu.async_copy(val_h.at[s, jsc], val2, stage_sem)
            a1.wait(); a2.wait(); a3.wait()
            for g in range(G):
                for i in range(C // LANES):
                    sl = pl.ds(i * LANES, LANES)
                    col2[g, sl] = col2[g, sl] + col_off

            gd = [None] * NBUF
            sd = [None] * NBUF
            gd[0] = pltpu.async_copy(src.at[col2.at[0]], gbufs[0], gsems[0])
            gd[1] = pltpu.async_copy(src.at[col2.at[1]], gbufs[1], gsems[1])
            for g in range(G):
                b = g % NBUF
                if g + 2 < G:
                    nb = (g + 2) % NBUF
                    if sd[nb] is not None:
                        sd[nb].wait()
                        sd[nb] = None
                    gd[nb] = pltpu.async_copy(src.at[col2.at[g + 2]],
                                              gbufs[nb], gsems[nb])
                gd[b].wait()

                def _scale(gi, cc, _g=g, _gb=gbufs[b]):
                    e0 = gi * LANES
                    v16 = val2[_g, pl.ds(e0, LANES)]
                    for j in range(LANES):
                        v = v16[j]
                        for d in range(HALF // LANES):
                            sl = pl.ds(d * LANES, LANES)
                            _gb[e0 + j, sl] = _gb[e0 + j, sl] * v
                    return cc

                lax.fori_loop(0, C // LANES, _scale, 0)
                sd[b] = pltpu.async_copy(gbufs[b], acc.at[row2.at[g]],
                                         ssems[b], add=True)
            for b in range(NBUF):
                if sd[b] is not None:
                    sd[b].wait()
            return carry

        lax.fori_loop(0, NSUPER, _super, 0)
        plsc.subcore_barrier()

        def _readout(q, carry):
            rb = base + q * RC
            pltpu.sync_copy(acc.at[pl.ds(rb, RC)], tmp)
            if k + 1 < N_LAYERS:
                dst = (egoa_h, egob_h)[k]
                pltpu.sync_copy(tmp, dst.at[pl.ds(c * N_PAD + rb, RC)])

            def _accum(r, carry2):
                for d in range(HALF // LANES):
                    sl = pl.ds(d * LANES, LANES)
                    sumv[q * RC + r, sl] = sumv[q * RC + r, sl] + tmp[r, sl]
                return carry2

            lax.fori_loop(0, RC, _accum, 0)
            lax.fori_loop(0, RC, _zero_tmp, 0)
            pltpu.sync_copy(tmp, acc.at[pl.ds(rb, RC)])
            return carry

        lax.fori_loop(0, RPT // RC, _readout, 0)
        plsc.subcore_barrier()

    inv = jnp.float32(1.0 / N_LAYERS)

    def _mean(r, carry):
        for d in range(HALF // LANES):
            sl = pl.ds(d * LANES, LANES)
            sumv[r, sl] = sumv[r, sl] * inv
        return carry

    lax.fori_loop(0, RPT, _mean, 0)
    pltpu.sync_copy(sumv, out_h.at[c, pl.ds(base, RPT)])


@jax.jit
def _run(ego0, col, row, val):
    mesh = plsc.VectorSubcoreMesh(core_axis_name="c", subcore_axis_name="s",
                                  num_cores=NC, num_subcores=NS)
    f = pl.kernel(
        _sc_body,
        out_type=(
            jax.ShapeDtypeStruct((NC, N_PAD, HALF), jnp.float32),
            jax.ShapeDtypeStruct((NC * N_PAD, HALF), jnp.float32),
            jax.ShapeDtypeStruct((NC * N_PAD, HALF), jnp.float32),
        ),
        mesh=mesh,
        compiler_params=pltpu.CompilerParams(use_tc_tiling_on_sc=False),
        scratch_types=[
            pltpu.VMEM((G, C), jnp.int32),
            pltpu.VMEM((G, C), jnp.int32),
            pltpu.VMEM((G, C), jnp.float32),
            [pltpu.VMEM((C, HALF), jnp.float32) for _ in range(NBUF)],
            pltpu.VMEM((RC, HALF), jnp.float32),
            pltpu.VMEM((RPT, HALF), jnp.float32),
            pltpu.VMEM_SHARED((N_PAD, HALF), jnp.float32),
            pltpu.SemaphoreType.DMA,
            [pltpu.SemaphoreType.DMA for _ in range(NBUF)],
            [pltpu.SemaphoreType.DMA for _ in range(NBUF)],
        ],
    )
    out, _, _ = f(ego0, col, row, val)
    return out


def kernel(user_emb, item_emb, edge_vals, edge_index):
    ego0 = jnp.concatenate([user_emb, item_emb], axis=0)
    ego0 = jnp.pad(ego0, ((0, N_PAD - N_NODES), (0, 0)))
    ego0 = ego0.reshape(N_PAD, NC, HALF).transpose(1, 0, 2).reshape(NC * N_PAD, HALF)

    pad = NS * EPS_PAD - E
    col = jnp.pad(edge_index[1], (0, pad)).reshape(NS, NSUPER, G, C)
    row = jnp.pad(edge_index[0], (0, pad)).reshape(NS, NSUPER, G, C)
    val = jnp.pad(edge_vals, (0, pad)).reshape(NS, NSUPER, G, C)

    out = _run(ego0, col, row, val)
    final = out[:, :N_NODES, :].transpose(1, 0, 2).reshape(N_NODES, D)
    return (final[:N_NODES // 2], final[N_NODES // 2:])


# speedup vs baseline: 7.2096x; 2.4991x over previous
import jax
import jax.numpy as jnp
from jax import lax
from jax.experimental import pallas as pl
from jax.experimental.pallas import tpu as pltpu
from jax.experimental.pallas import tpu_sc as plsc

N_NODES = 10000
D = 128
E = 320000
N_LAYERS = 3

NC = 2
NS = 16
HALF = D // NC
RPT = 640
N_PAD = NS * RPT
C = 128
G = 8
NBUF = 3
EPS = E // NS
NSUPER = -(-EPS // (G * C))
EPS_PAD = NSUPER * G * C
LANES = 16
RC = 128


def _sc_body(ego0_h, col_h, row_h, val_h, out_h,
             col2, row2, val2, gbufs, tmp, tmp2, ego_sp, acc,
             stage_sem, gsems, ssems):
    c = lax.axis_index("c")
    s = lax.axis_index("s")
    base = s * RPT
    zero16f = jnp.zeros((LANES,), jnp.float32)

    def _zero_tmp2(r, carry):
        for d in range(HALF // LANES):
            tmp2[r, pl.ds(d * LANES, LANES)] = zero16f
        return carry

    lax.fori_loop(0, RC, _zero_tmp2, 0)

    def _init(q, carry):
        rb = base + q * RC
        pltpu.sync_copy(ego0_h.at[pl.ds(c * N_PAD + rb, RC)], tmp)
        pltpu.sync_copy(tmp, ego_sp.at[pl.ds(rb, RC)])
        pltpu.sync_copy(tmp2, acc.at[pl.ds(rb, RC)])
        return carry

    lax.fori_loop(0, RPT // RC, _init, 0)
    plsc.subcore_barrier()

    for k in range(N_LAYERS):
        def _super(jsc, carry):
            a1 = pltpu.async_copy(col_h.at[s, jsc], col2, stage_sem)
            a2 = pltpu.async_copy(row_h.at[s, jsc], row2, stage_sem)
            a3 = pltpu.async_copy(val_h.at[s, jsc], val2, stage_sem)
            a1.wait(); a2.wait(); a3.wait()

            gd = [None] * NBUF
            sd = [None] * NBUF
            gd[0] = pltpu.async_copy(ego_sp.at[col2.at[0]], gbufs[0],
                                     gsems[0])
            for g in range(G):
                b = g % NBUF
                if g + 1 < G:
                    nb = (g + 1) % NBUF
                    if sd[nb] is not None:
                        sd[nb].wait()
                        sd[nb] = None
                    gd[nb] = pltpu.async_copy(ego_sp.at[col2.at[g + 1]],
                                              gbufs[nb], gsems[nb])
                gd[b].wait()
                gb = gbufs[b]

                @plsc.parallel_loop(0, C // LANES, unroll=2)
                def _scale(gi, _g=g, _gb=gb):
                    e0 = gi * LANES
                    v16 = val2[_g, pl.ds(e0, LANES)]
                    for j in range(LANES):
                        v = v16[j]
                        for d in range(HALF // LANES):
                            sl = pl.ds(d * LANES, LANES)
                            _gb[e0 + j, sl] = _gb[e0 + j, sl] * v

                sd[b] = pltpu.async_copy(gb, acc.at[row2.at[g]],
                                         ssems[b], add=True)
            for b in range(NBUF):
                if sd[b] is not None:
                    sd[b].wait()
            return carry

        lax.fori_loop(0, NSUPER, _super, 0)
        plsc.subcore_barrier()

        def _readout(q, carry):
            rb = base + q * RC
            pltpu.sync_copy(acc.at[pl.ds(rb, RC)], tmp)
            if k == 0:
                pltpu.sync_copy(tmp, ego_sp.at[pl.ds(rb, RC)])
            elif k + 1 < N_LAYERS:
                pltpu.sync_copy(ego_sp.at[pl.ds(rb, RC)], tmp2)

                @plsc.parallel_loop(0, RC, unroll=4)
                def _diff(r):
                    for d in range(HALF // LANES):
                        sl = pl.ds(d * LANES, LANES)
                        tmp2[r, sl] = tmp[r, sl] - tmp2[r, sl]

                pltpu.sync_copy(tmp2, ego_sp.at[pl.ds(rb, RC)])
            else:
                inv = jnp.float32(1.0 / N_LAYERS)

                @plsc.parallel_loop(0, RC, unroll=4)
                def _mean(r):
                    for d in range(HALF // LANES):
                        sl = pl.ds(d * LANES, LANES)
                        tmp[r, sl] = tmp[r, sl] * inv

                pltpu.sync_copy(tmp, out_h.at[c, pl.ds(rb, RC)])
            return carry

        lax.fori_loop(0, RPT // RC, _readout, 0)
        if k + 1 < N_LAYERS:
            plsc.subcore_barrier()


@jax.jit
def _run(ego0, col, row, val):
    mesh = plsc.VectorSubcoreMesh(core_axis_name="c", subcore_axis_name="s",
                                  num_cores=NC, num_subcores=NS)
    f = pl.kernel(
        _sc_body,
        out_type=jax.ShapeDtypeStruct((NC, N_PAD, HALF), jnp.float32),
        mesh=mesh,
        compiler_params=pltpu.CompilerParams(use_tc_tiling_on_sc=False),
        scratch_types=[
            pltpu.VMEM((G, C), jnp.int32),
            pltpu.VMEM((G, C), jnp.int32),
            pltpu.VMEM((G, C), jnp.float32),
            [pltpu.VMEM((C, HALF), jnp.float32) for _ in range(NBUF)],
            pltpu.VMEM((RC, HALF), jnp.float32),
            pltpu.VMEM((RC, HALF), jnp.float32),
            pltpu.VMEM_SHARED((N_PAD, HALF), jnp.float32),
            pltpu.VMEM_SHARED((N_PAD, HALF), jnp.float32),
            pltpu.SemaphoreType.DMA,
            [pltpu.SemaphoreType.DMA for _ in range(NBUF)],
            [pltpu.SemaphoreType.DMA for _ in range(NBUF)],
        ],
    )
    return f(ego0, col, row, val)


def kernel(user_emb, item_emb, edge_vals, edge_index):
    ego0 = jnp.concatenate([user_emb, item_emb], axis=0)
    ego0 = jnp.pad(ego0, ((0, N_PAD - N_NODES), (0, 0)))
    ego0 = ego0.reshape(N_PAD, NC, HALF).transpose(1, 0, 2).reshape(NC * N_PAD, HALF)

    pad = NS * EPS_PAD - E
    col = jnp.pad(edge_index[1], (0, pad)).reshape(NS, NSUPER, G, C)
    row = jnp.pad(edge_index[0], (0, pad)).reshape(NS, NSUPER, G, C)
    val = jnp.pad(edge_vals, (0, pad)).reshape(NS, NSUPER, G, C)

    out = _run(ego0, col, row, val)
    final = out[:, :N_NODES, :].transpose(1, 0, 2).reshape(N_NODES, D)
    return (final[:N_NODES // 2], final[N_NODES // 2:])


# speedup vs baseline: 7.4720x; 1.0364x over previous
import jax
import jax.numpy as jnp
from jax import lax
from jax.experimental import pallas as pl
from jax.experimental.pallas import tpu as pltpu
from jax.experimental.pallas import tpu_sc as plsc

N_NODES = 10000
D = 128
E = 320000
N_LAYERS = 3

NC = 2
NS = 16
HALF = D // NC
RPT = 640
N_PAD = NS * RPT
C = 128
G = 8
NBUF = 4
EPS = E // NS
NSUPER = -(-EPS // (G * C))
EPS_PAD = NSUPER * G * C
LANES = 16
RC = 128


def _sc_body(ego0_h, col_h, row_h, val_h, out_h,
             col2, row2, val2, gbufs, ego_sp, acc,
             stage_sem, gsems, ssems):
    c = lax.axis_index("c")
    s = lax.axis_index("s")
    base = s * RPT
    zero16f = jnp.zeros((LANES,), jnp.float32)
    tmp, tmp2 = gbufs[0], gbufs[1]

    def _zero_tmp2(r, carry):
        for d in range(HALF // LANES):
            tmp2[r, pl.ds(d * LANES, LANES)] = zero16f
        return carry

    lax.fori_loop(0, RC, _zero_tmp2, 0)

    def _init(q, carry):
        rb = base + q * RC
        pltpu.sync_copy(ego0_h.at[pl.ds(c * N_PAD + rb, RC)], tmp)
        pltpu.sync_copy(tmp, ego_sp.at[pl.ds(rb, RC)])
        pltpu.sync_copy(tmp2, acc.at[pl.ds(rb, RC)])
        return carry

    lax.fori_loop(0, RPT // RC, _init, 0)
    plsc.subcore_barrier()

    for k in range(N_LAYERS):
        def _super(jsc, carry):
            a1 = pltpu.async_copy(col_h.at[s, jsc], col2, stage_sem)
            a2 = pltpu.async_copy(row_h.at[s, jsc], row2, stage_sem)
            a3 = pltpu.async_copy(val_h.at[s, jsc], val2, stage_sem)
            a1.wait(); a2.wait(); a3.wait()

            gd = [None] * NBUF
            sd = [None] * NBUF
            gd[0] = pltpu.async_copy(ego_sp.at[col2.at[0]], gbufs[0],
                                     gsems[0])
            gd[1] = pltpu.async_copy(ego_sp.at[col2.at[1]], gbufs[1],
                                     gsems[1])
            for g in range(G):
                b = g % NBUF
                if g + 2 < G:
                    nb = (g + 2) % NBUF
                    if sd[nb] is not None:
                        sd[nb].wait()
                        sd[nb] = None
                    gd[nb] = pltpu.async_copy(ego_sp.at[col2.at[g + 2]],
                                              gbufs[nb], gsems[nb])
                gd[b].wait()
                gb = gbufs[b]

                @plsc.parallel_loop(0, C // LANES, unroll=2)
                def _scale(gi, _g=g, _gb=gb):
                    e0 = gi * LANES
                    v16 = val2[_g, pl.ds(e0, LANES)]
                    for j in range(LANES):
                        v = v16[j]
                        for d in range(HALF // LANES):
                            sl = pl.ds(d * LANES, LANES)
                            _gb[e0 + j, sl] = _gb[e0 + j, sl] * v

                sd[b] = pltpu.async_copy(gb, acc.at[row2.at[g]],
                                         ssems[b], add=True)
            for b in range(NBUF):
                if sd[b] is not None:
                    sd[b].wait()
            return carry

        lax.fori_loop(0, NSUPER, _super, 0)
        plsc.subcore_barrier()

        def _readout(q, carry):
            rb = base + q * RC
            pltpu.sync_copy(acc.at[pl.ds(rb, RC)], tmp)
            if k == 0:
                pltpu.sync_copy(tmp, ego_sp.at[pl.ds(rb, RC)])
            elif k + 1 < N_LAYERS:
                pltpu.sync_copy(ego_sp.at[pl.ds(rb, RC)], tmp2)

                @plsc.parallel_loop(0, RC, unroll=4)
                def _diff(r):
                    for d in range(HALF // LANES):
                        sl = pl.ds(d * LANES, LANES)
                        tmp2[r, sl] = tmp[r, sl] - tmp2[r, sl]

                pltpu.sync_copy(tmp2, ego_sp.at[pl.ds(rb, RC)])
            else:
                inv = jnp.float32(1.0 / N_LAYERS)

                @plsc.parallel_loop(0, RC, unroll=4)
                def _mean(r):
                    for d in range(HALF // LANES):
                        sl = pl.ds(d * LANES, LANES)
                        tmp[r, sl] = tmp[r, sl] * inv

                pltpu.sync_copy(tmp, out_h.at[c, pl.ds(rb, RC)])
            return carry

        lax.fori_loop(0, RPT // RC, _readout, 0)
        if k + 1 < N_LAYERS:
            plsc.subcore_barrier()


@jax.jit
def _run(ego0, col, row, val):
    mesh = plsc.VectorSubcoreMesh(core_axis_name="c", subcore_axis_name="s",
                                  num_cores=NC, num_subcores=NS)
    f = pl.kernel(
        _sc_body,
        out_type=jax.ShapeDtypeStruct((NC, N_PAD, HALF), jnp.float32),
        mesh=mesh,
        compiler_params=pltpu.CompilerParams(use_tc_tiling_on_sc=False),
        scratch_types=[
            pltpu.VMEM((G, C), jnp.int32),
            pltpu.VMEM((G, C), jnp.int32),
            pltpu.VMEM((G, C), jnp.float32),
            [pltpu.VMEM((C, HALF), jnp.float32) for _ in range(NBUF)],
            pltpu.VMEM_SHARED((N_PAD, HALF), jnp.float32),
            pltpu.VMEM_SHARED((N_PAD, HALF), jnp.float32),
            pltpu.SemaphoreType.DMA,
            [pltpu.SemaphoreType.DMA for _ in range(NBUF)],
            [pltpu.SemaphoreType.DMA for _ in range(NBUF)],
        ],
    )
    return f(ego0, col, row, val)


def kernel(user_emb, item_emb, edge_vals, edge_index):
    ego0 = jnp.concatenate([user_emb, item_emb], axis=0)
    ego0 = jnp.pad(ego0, ((0, N_PAD - N_NODES), (0, 0)))
    ego0 = ego0.reshape(N_PAD, NC, HALF).transpose(1, 0, 2).reshape(NC * N_PAD, HALF)

    pad = NS * EPS_PAD - E
    col = jnp.pad(edge_index[1], (0, pad)).reshape(NS, NSUPER, G, C)
    row = jnp.pad(edge_index[0], (0, pad)).reshape(NS, NSUPER, G, C)
    val = jnp.pad(edge_vals, (0, pad)).reshape(NS, NSUPER, G, C)

    out = _run(ego0, col, row, val)
    final = out[:, :N_NODES, :].transpose(1, 0, 2).reshape(N_NODES, D)
    return (final[:N_NODES // 2], final[N_NODES // 2:])
